# baseline (speedup 1.0000x reference)
import jax
import jax.numpy as jnp
from jax import lax
from jax.experimental import pallas as pl
from jax.experimental.pallas import tpu as pltpu

B, P, K = 32, 128, 16
G = 16
_KCHUNKS = 16
EPS = 1e-5
_HI = lax.Precision.HIGHEST
_MED = lax.Precision.DEFAULT


def _edge_conv(h, wd, wb, b1, w2, b2):
    cin = h.shape[-1]
    cout = w2.shape[1]
    sq = jnp.sum(h * h, axis=2, keepdims=True)
    hf = h.reshape(G * P, cin)
    a = (jnp.dot(hf, wd, precision=_MED) + b1).reshape(G, P, cout)
    bv = jnp.dot(hf, wb, precision=_MED).reshape(G, P, cout)

    d = jnp.concatenate(
        [(sq[g] + sq[g].T
          - 2.0 * lax.dot_general(h[g], h[g], (((1,), (1,)), ((), ())),
                                  precision=_HI))[None]
         for g in range(G)], axis=0)

    col = lax.broadcasted_iota(jnp.int32, (G, P, P), 2).astype(jnp.float32)

    af = a.reshape(G * P, cout)
    dw = d
    out = None
    for k in range(K):
        m = jnp.min(dw, axis=2, keepdims=True)
        eq = dw == m
        first = jnp.min(jnp.where(eq, col, jnp.float32(P)), axis=2,
                        keepdims=True)
        ohb = col == first
        oh = ohb.astype(jnp.float32)
        dw = jnp.where(ohb, jnp.float32(jnp.inf), dw)
        gk = jnp.concatenate(
            [jnp.dot(oh[g], bv[g], precision=_MED) for g in range(G)],
            axis=0)
        r1 = jnp.maximum(af + gk, 0.0)
        h2 = jnp.dot(r1, w2, precision=_MED) + b2
        out = jnp.maximum(h2, 0.0) if out is None else jnp.maximum(out, h2)
    return out.reshape(G, P, cout)


def _pnet_kernel(x_ref,
                 wd0, wb0, b10, w20, b20,
                 wd1, wb1, b11, w21, b21,
                 wd2, wb2, b12, w22, b22,
                 wc1, bc1, wc2, bc2, wc3, bc3,
                 out_ref):
    h = x_ref[...]
    h = _edge_conv(h, wd0[...], wb0[...], b10[...], w20[...], b20[...])
    h = _edge_conv(h, wd1[...], wb1[...], b11[...], w21[...], b21[...])
    h = _edge_conv(h, wd2[...], wb2[...], b12[...], w22[...], b22[...])
    mean = jnp.mean(h, axis=1)
    mx = jnp.max(h, axis=1)
    z = jnp.concatenate([mean, mx], axis=1)
    z = jnp.maximum(jnp.dot(z, wc1[...], precision=_HI) + bc1[...], 0.0)
    z = jnp.maximum(jnp.dot(z, wc2[...], precision=_HI) + bc2[...], 0.0)
    out_ref[...] = (jnp.dot(z, wc3[...], precision=_HI) + bc3[...])[:, None]


def _fold_edge(p):
    s1 = p["g1"] / jnp.sqrt(1.0 + EPS)
    w1f = p["W1"] * s1[None, :]
    b1f = p["b1"] * s1 + p["be1"]
    c = p["W1"].shape[0] // 2
    w1a, w1b = w1f[:c], w1f[c:]
    s2 = p["g2"] / jnp.sqrt(1.0 + EPS)
    w2f = p["W2"] * s2[None, :]
    b2f = p["b2"] * s2 + p["be2"]
    return w1a - w1b, w1b, b1f[None, :], w2f, b2f[None, :]


def kernel(x, batch, params):
    xb = jnp.pad(x.reshape(B, P, 6), ((0, 0), (0, 0), (0, 2)))

    wd0, wb0, b10, w20, b20 = _fold_edge(params["conv0"])
    wd0 = jnp.pad(wd0, ((0, 2), (0, 0)))
    wb0 = jnp.pad(wb0, ((0, 2), (0, 0)))
    wd1, wb1, b11, w21, b21 = _fold_edge(params["conv1"])
    wd2, wb2, b12, w22, b22 = _fold_edge(params["conv2"])

    c = params["cls"]
    s1 = c["g1"] / jnp.sqrt(1.0 + EPS)
    wc1 = c["W1"] * s1[None, :]
    bc1 = (c["b1"] * s1 + c["be1"])[None, :]
    s2 = c["g2"] / jnp.sqrt(1.0 + EPS)
    wc2 = c["W2"] * s2[None, :]
    bc2 = (c["b2"] * s2 + c["be2"])[None, :]
    wc3 = jnp.pad(c["W3"], ((0, 0), (0, 126)))
    bc3 = jnp.pad(c["b3"], (0, 126))[None, :]

    ops = [wd0, wb0, b10, w20, b20,
           wd1, wb1, b11, w21, b21,
           wd2, wb2, b12, w22, b22,
           wc1, bc1, wc2, bc2, wc3, bc3]

    def full(a):
        return pl.BlockSpec(a.shape, lambda i: (0,) * a.ndim)

    out = pl.pallas_call(
        _pnet_kernel,
        grid=(B // G,),
        in_specs=[pl.BlockSpec((G, P, 8), lambda i: (i, 0, 0))] +
                 [full(a) for a in ops],
        out_specs=pl.BlockSpec((G, 1, 128), lambda i: (i, 0, 0)),
        out_shape=jax.ShapeDtypeStruct((B, 1, 128), jnp.float32),
    )(xb, *ops)
    return out.reshape(B, 128)[:, :2]


# speedup vs baseline: 1.2470x; 1.2470x over previous
import jax
import jax.numpy as jnp
from jax import lax
from jax.experimental import pallas as pl
from jax.experimental.pallas import tpu as pltpu

B, P, K = 32, 128, 16
G = 8
_KCHUNKS = 16
EPS = 1e-5
_HI = lax.Precision.HIGHEST
_MED = lax.Precision.DEFAULT


def _edge_conv(h, wd, wb, b1, w2, b2):
    cin = h.shape[-1]
    cout = w2.shape[1]
    sq = jnp.sum(h * h, axis=2, keepdims=True)
    hf = h.reshape(G * P, cin)
    a = (jnp.dot(hf, wd, precision=_MED) + b1).reshape(G, P, cout)
    bv = jnp.dot(hf, wb, precision=_MED).reshape(G, P, cout)

    d = jnp.concatenate(
        [(sq[g] + sq[g].T
          - 2.0 * lax.dot_general(h[g], h[g], (((1,), (1,)), ((), ())),
                                  precision=_HI))[None]
         for g in range(G)], axis=0)

    col = lax.broadcasted_iota(jnp.int32, (G, P, P), 2).astype(jnp.float32)

    af = a.reshape(G * P, cout)
    dw = d
    out = None
    for k in range(K):
        m = jnp.min(dw, axis=2, keepdims=True)
        eq = dw == m
        first = jnp.min(jnp.where(eq, col, jnp.float32(P)), axis=2,
                        keepdims=True)
        ohb = col == first
        oh = ohb.astype(jnp.float32)
        dw = jnp.where(ohb, jnp.float32(jnp.inf), dw)
        gk = lax.dot_general(oh, bv, (((2,), (1,)), ((0,), (0,))),
                             precision=_MED).reshape(G * P, cout)
        r1 = jnp.maximum(af + gk, 0.0)
        h2 = jnp.dot(r1, w2, precision=_MED) + b2
        out = jnp.maximum(h2, 0.0) if out is None else jnp.maximum(out, h2)
    return out.reshape(G, P, cout)


def _pnet_kernel(x_ref,
                 wd0, wb0, b10, w20, b20,
                 wd1, wb1, b11, w21, b21,
                 wd2, wb2, b12, w22, b22,
                 wc1, bc1, wc2, bc2, wc3, bc3,
                 out_ref):
    h = x_ref[...]
    h = _edge_conv(h, wd0[...], wb0[...], b10[...], w20[...], b20[...])
    h = _edge_conv(h, wd1[...], wb1[...], b11[...], w21[...], b21[...])
    h = _edge_conv(h, wd2[...], wb2[...], b12[...], w22[...], b22[...])
    mean = jnp.mean(h, axis=1)
    mx = jnp.max(h, axis=1)
    z = jnp.concatenate([mean, mx], axis=1)
    z = jnp.maximum(jnp.dot(z, wc1[...], precision=_HI) + bc1[...], 0.0)
    z = jnp.maximum(jnp.dot(z, wc2[...], precision=_HI) + bc2[...], 0.0)
    out_ref[...] = (jnp.dot(z, wc3[...], precision=_HI) + bc3[...])[:, None]


def _fold_edge(p):
    s1 = p["g1"] / jnp.sqrt(1.0 + EPS)
    w1f = p["W1"] * s1[None, :]
    b1f = p["b1"] * s1 + p["be1"]
    c = p["W1"].shape[0] // 2
    w1a, w1b = w1f[:c], w1f[c:]
    s2 = p["g2"] / jnp.sqrt(1.0 + EPS)
    w2f = p["W2"] * s2[None, :]
    b2f = p["b2"] * s2 + p["be2"]
    return w1a - w1b, w1b, b1f[None, :], w2f, b2f[None, :]


def kernel(x, batch, params):
    xb = jnp.pad(x.reshape(B, P, 6), ((0, 0), (0, 0), (0, 2)))

    wd0, wb0, b10, w20, b20 = _fold_edge(params["conv0"])
    wd0 = jnp.pad(wd0, ((0, 2), (0, 0)))
    wb0 = jnp.pad(wb0, ((0, 2), (0, 0)))
    wd1, wb1, b11, w21, b21 = _fold_edge(params["conv1"])
    wd2, wb2, b12, w22, b22 = _fold_edge(params["conv2"])

    c = params["cls"]
    s1 = c["g1"] / jnp.sqrt(1.0 + EPS)
    wc1 = c["W1"] * s1[None, :]
    bc1 = (c["b1"] * s1 + c["be1"])[None, :]
    s2 = c["g2"] / jnp.sqrt(1.0 + EPS)
    wc2 = c["W2"] * s2[None, :]
    bc2 = (c["b2"] * s2 + c["be2"])[None, :]
    wc3 = jnp.pad(c["W3"], ((0, 0), (0, 126)))
    bc3 = jnp.pad(c["b3"], (0, 126))[None, :]

    ops = [wd0, wb0, b10, w20, b20,
           wd1, wb1, b11, w21, b21,
           wd2, wb2, b12, w22, b22,
           wc1, bc1, wc2, bc2, wc3, bc3]

    def full(a):
        return pl.BlockSpec(a.shape, lambda i: (0,) * a.ndim)

    out = pl.pallas_call(
        _pnet_kernel,
        grid=(B // G,),
        in_specs=[pl.BlockSpec((G, P, 8), lambda i: (i, 0, 0))] +
                 [full(a) for a in ops],
        out_specs=pl.BlockSpec((G, 1, 128), lambda i: (i, 0, 0)),
        out_shape=jax.ShapeDtypeStruct((B, 1, 128), jnp.float32),
    )(xb, *ops)
    return out.reshape(B, 128)[:, :2]


# speedup vs baseline: 1.2617x; 1.0117x over previous
import jax
import jax.numpy as jnp
from jax import lax
from jax.experimental import pallas as pl
from jax.experimental.pallas import tpu as pltpu

B, P, K = 32, 128, 16
G = 8
_KCHUNKS = 16
EPS = 1e-5
_HI = lax.Precision.HIGHEST
_MED = lax.Precision.DEFAULT


def _edge_conv(h, wd, wb, b1, w2, b2):
    cin = h.shape[-1]
    cout = w2.shape[1]
    sq = jnp.sum(h * h, axis=2, keepdims=True)
    hf = h.reshape(G * P, cin)
    a = (jnp.dot(hf, wd, precision=_MED) + b1).reshape(G, P, cout)
    bv = jnp.dot(hf, wb, precision=_MED).reshape(G, P, cout)

    gram = lax.dot_general(h, h, (((2,), (2,)), ((0,), (0,))),
                           precision=_HI)
    d = sq + sq.reshape(G, 1, P) - 2.0 * gram

    col = lax.broadcasted_iota(jnp.int32, (G, P, P), 2).astype(jnp.float32)

    af = a.reshape(G * P, cout)
    dw = d
    out = None
    for k in range(K):
        m = jnp.min(dw, axis=2, keepdims=True)
        eq = dw == m
        first = jnp.min(jnp.where(eq, col, jnp.float32(P)), axis=2,
                        keepdims=True)
        ohb = col == first
        oh = ohb.astype(jnp.float32)
        dw = jnp.where(ohb, jnp.float32(jnp.inf), dw)
        gk = lax.dot_general(oh, bv, (((2,), (1,)), ((0,), (0,))),
                             precision=_MED).reshape(G * P, cout)
        r1 = jnp.maximum(af + gk, 0.0)
        h2 = jnp.dot(r1, w2, precision=_MED) + b2
        out = jnp.maximum(h2, 0.0) if out is None else jnp.maximum(out, h2)
    return out.reshape(G, P, cout)


def _pnet_kernel(x_ref,
                 wd0, wb0, b10, w20, b20,
                 wd1, wb1, b11, w21, b21,
                 wd2, wb2, b12, w22, b22,
                 wc1, bc1, wc2, bc2, wc3, bc3,
                 out_ref):
    h = x_ref[...]
    h = _edge_conv(h, wd0[...], wb0[...], b10[...], w20[...], b20[...])
    h = _edge_conv(h, wd1[...], wb1[...], b11[...], w21[...], b21[...])
    h = _edge_conv(h, wd2[...], wb2[...], b12[...], w22[...], b22[...])
    mean = jnp.mean(h, axis=1)
    mx = jnp.max(h, axis=1)
    z = jnp.concatenate([mean, mx], axis=1)
    z = jnp.maximum(jnp.dot(z, wc1[...], precision=_HI) + bc1[...], 0.0)
    z = jnp.maximum(jnp.dot(z, wc2[...], precision=_HI) + bc2[...], 0.0)
    out_ref[...] = (jnp.dot(z, wc3[...], precision=_HI) + bc3[...])[:, None]


def _fold_edge(p):
    s1 = p["g1"] / jnp.sqrt(1.0 + EPS)
    w1f = p["W1"] * s1[None, :]
    b1f = p["b1"] * s1 + p["be1"]
    c = p["W1"].shape[0] // 2
    w1a, w1b = w1f[:c], w1f[c:]
    s2 = p["g2"] / jnp.sqrt(1.0 + EPS)
    w2f = p["W2"] * s2[None, :]
    b2f = p["b2"] * s2 + p["be2"]
    return w1a - w1b, w1b, b1f[None, :], w2f, b2f[None, :]


def kernel(x, batch, params):
    xb = jnp.pad(x.reshape(B, P, 6), ((0, 0), (0, 0), (0, 2)))

    wd0, wb0, b10, w20, b20 = _fold_edge(params["conv0"])
    wd0 = jnp.pad(wd0, ((0, 2), (0, 0)))
    wb0 = jnp.pad(wb0, ((0, 2), (0, 0)))
    wd1, wb1, b11, w21, b21 = _fold_edge(params["conv1"])
    wd2, wb2, b12, w22, b22 = _fold_edge(params["conv2"])

    c = params["cls"]
    s1 = c["g1"] / jnp.sqrt(1.0 + EPS)
    wc1 = c["W1"] * s1[None, :]
    bc1 = (c["b1"] * s1 + c["be1"])[None, :]
    s2 = c["g2"] / jnp.sqrt(1.0 + EPS)
    wc2 = c["W2"] * s2[None, :]
    bc2 = (c["b2"] * s2 + c["be2"])[None, :]
    wc3 = jnp.pad(c["W3"], ((0, 0), (0, 126)))
    bc3 = jnp.pad(c["b3"], (0, 126))[None, :]

    ops = [wd0, wb0, b10, w20, b20,
           wd1, wb1, b11, w21, b21,
           wd2, wb2, b12, w22, b22,
           wc1, bc1, wc2, bc2, wc3, bc3]

    def full(a):
        return pl.BlockSpec(a.shape, lambda i: (0,) * a.ndim)

    out = pl.pallas_call(
        _pnet_kernel,
        grid=(B // G,),
        in_specs=[pl.BlockSpec((G, P, 8), lambda i: (i, 0, 0))] +
                 [full(a) for a in ops],
        out_specs=pl.BlockSpec((G, 1, 128), lambda i: (i, 0, 0)),
        out_shape=jax.ShapeDtypeStruct((B, 1, 128), jnp.float32),
    )(xb, *ops)
    return out.reshape(B, 128)[:, :2]


# speedup vs baseline: 1.2625x; 1.0006x over previous
import jax
import jax.numpy as jnp
from jax import lax
from jax.experimental import pallas as pl

B, P, K = 32, 128, 16
G = 8
EPS = 1e-5
_HI = lax.Precision.HIGHEST
_MED = lax.Precision.DEFAULT


def _edge_conv(h, wd, wb, b1, w2, b2):
    cin = h.shape[-1]
    cout = w2.shape[1]
    sq = jnp.sum(h * h, axis=2, keepdims=True)
    hf = h.reshape(G * P, cin)
    a = (jnp.dot(hf, wd, precision=_MED) + b1).reshape(G, P, cout)
    bv = jnp.dot(hf, wb, precision=_MED).reshape(G, P, cout)

    gram = lax.dot_general(h, h, (((2,), (2,)), ((0,), (0,))),
                           precision=_HI)
    d = sq + sq.reshape(G, 1, P) - 2.0 * gram

    col = lax.broadcasted_iota(jnp.int32, (G, P, P), 2).astype(jnp.float32)

    af = a.reshape(G * P, cout)
    dw = d
    out = None
    for k in range(K):
        m = jnp.min(dw, axis=2, keepdims=True)
        eq = dw == m
        first = jnp.min(jnp.where(eq, col, jnp.float32(P)), axis=2,
                        keepdims=True)
        ohb = col == first
        oh = ohb.astype(jnp.float32)
        dw = jnp.where(ohb, jnp.float32(jnp.inf), dw)
        gk = lax.dot_general(oh, bv, (((2,), (1,)), ((0,), (0,))),
                             precision=_MED).reshape(G * P, cout)
        r1 = jnp.maximum(af + gk, 0.0)
        h2 = jnp.dot(r1, w2, precision=_MED) + b2
        out = jnp.maximum(h2, 0.0) if out is None else jnp.maximum(out, h2)
    return out.reshape(G, P, cout)


def _pnet_kernel(x_ref,
                 wd0, wb0, b10, w20, b20,
                 wd1, wb1, b11, w21, b21,
                 wd2, wb2, b12, w22, b22,
                 wc1, bc1, wc2, bc2, wc3, bc3,
                 out_ref):
    h = x_ref[...]
    h = _edge_conv(h, wd0[...], wb0[...], b10[...], w20[...], b20[...])
    h = _edge_conv(h, wd1[...], wb1[...], b11[...], w21[...], b21[...])
    h = _edge_conv(h, wd2[...], wb2[...], b12[...], w22[...], b22[...])
    mean = jnp.mean(h, axis=1)
    mx = jnp.max(h, axis=1)
    z = jnp.concatenate([mean, mx], axis=1)
    z = jnp.maximum(jnp.dot(z, wc1[...], precision=_HI) + bc1[...], 0.0)
    z = jnp.maximum(jnp.dot(z, wc2[...], precision=_HI) + bc2[...], 0.0)
    out_ref[...] = (jnp.dot(z, wc3[...], precision=_HI) + bc3[...])[:, None]


def _fold_edge(p):
    s1 = p["g1"] / jnp.sqrt(1.0 + EPS)
    w1f = p["W1"] * s1[None, :]
    b1f = p["b1"] * s1 + p["be1"]
    c = p["W1"].shape[0] // 2
    w1a, w1b = w1f[:c], w1f[c:]
    s2 = p["g2"] / jnp.sqrt(1.0 + EPS)
    w2f = p["W2"] * s2[None, :]
    b2f = p["b2"] * s2 + p["be2"]
    return w1a - w1b, w1b, b1f[None, :], w2f, b2f[None, :]


def kernel(x, batch, params):
    xb = jnp.pad(x.reshape(B, P, 6), ((0, 0), (0, 0), (0, 2)))

    wd0, wb0, b10, w20, b20 = _fold_edge(params["conv0"])
    wd0 = jnp.pad(wd0, ((0, 2), (0, 0)))
    wb0 = jnp.pad(wb0, ((0, 2), (0, 0)))
    wd1, wb1, b11, w21, b21 = _fold_edge(params["conv1"])
    wd2, wb2, b12, w22, b22 = _fold_edge(params["conv2"])

    c = params["cls"]
    s1 = c["g1"] / jnp.sqrt(1.0 + EPS)
    wc1 = c["W1"] * s1[None, :]
    bc1 = (c["b1"] * s1 + c["be1"])[None, :]
    s2 = c["g2"] / jnp.sqrt(1.0 + EPS)
    wc2 = c["W2"] * s2[None, :]
    bc2 = (c["b2"] * s2 + c["be2"])[None, :]
    wc3 = jnp.pad(c["W3"], ((0, 0), (0, 126)))
    bc3 = jnp.pad(c["b3"], (0, 126))[None, :]

    ops = [wd0, wb0, b10, w20, b20,
           wd1, wb1, b11, w21, b21,
           wd2, wb2, b12, w22, b22,
           wc1, bc1, wc2, bc2, wc3, bc3]

    def full(a):
        return pl.BlockSpec(a.shape, lambda i: (0,) * a.ndim)

    out = pl.pallas_call(
        _pnet_kernel,
        grid=(B // G,),
        in_specs=[pl.BlockSpec((G, P, 8), lambda i: (i, 0, 0))] +
                 [full(a) for a in ops],
        out_specs=pl.BlockSpec((G, 1, 128), lambda i: (i, 0, 0)),
        out_shape=jax.ShapeDtypeStruct((B, 1, 128), jnp.float32),
    )(xb, *ops)
    return out.reshape(B, 128)[:, :2]
